# baseline (speedup 1.0000x reference)
import functools

import jax
import jax.numpy as jnp
from jax import lax
from jax.experimental import pallas as pl
from jax.experimental.pallas import tpu as pltpu
from jax.experimental.pallas import tpu_sc as plsc

_NUM_CORES = 2
_CHUNK_ROWS = 512


@functools.lru_cache(maxsize=None)
def _make_copy_kernel(seq_len: int, dim: int):
    rows_per_c = seq_len // _NUM_CORES
    chunk = min(rows_per_c, _CHUNK_ROWS)
    nchunk = rows_per_c // chunk
    mesh = plsc.ScalarSubcoreMesh(axis_name="c", num_cores=_NUM_CORES)

    @functools.partial(
        pl.kernel,
        mesh=mesh,
        out_type=jax.ShapeDtypeStruct((seq_len, dim), jnp.float32),
        scratch_types=[
            pltpu.VMEM_SHARED((2, chunk, dim), jnp.float32),
            pltpu.SemaphoreType.DMA,
            pltpu.SemaphoreType.DMA,
            pltpu.SemaphoreType.DMA,
            pltpu.SemaphoreType.DMA,
        ],
    )
    def k(emb_hbm, out_hbm, shared, rsem0, rsem1, wsem0, wsem1):
        rsems = (rsem0, rsem1)
        wsems = (wsem0, wsem1)
        base = lax.axis_index("c") * rows_per_c

        def read(c):
            b = c % 2
            return pltpu.async_copy(
                emb_hbm.at[pl.ds(base + c * chunk, chunk)],
                shared.at[b], rsems[b])

        def write(c):
            b = c % 2
            return pltpu.async_copy(
                shared.at[b],
                out_hbm.at[pl.ds(base + c * chunk, chunk)], wsems[b])

        reads = {0: read(0)}
        writes = {}
        for c in range(nchunk):
            if c + 1 < nchunk:
                if c - 1 >= 0:
                    writes.pop(c - 1).wait()
                reads[c + 1] = read(c + 1)
            reads.pop(c).wait()
            writes[c] = write(c)
        for w in writes.values():
            w.wait()

    return k


def kernel(x, emb_weight):
    seq_len = x.shape[1]
    dim = emb_weight.shape[1]
    return _make_copy_kernel(seq_len, dim)(emb_weight)
